# speedup vs baseline: 5.2082x; 5.2082x over previous
import functools

import jax
import jax.numpy as jnp
from jax import lax
from jax.experimental import pallas as pl
from jax.experimental.pallas import tpu as pltpu
from jax.experimental.pallas import tpu_sc as plsc

N_NODES = 10000
N_EDGES = 320000
D = 128

NC = 2
NS = 16
NW = NC * NS
EPW = N_EDGES // NW
CH = 200
NCH = EPW // CH
SUB = 100
NSUB = CH // SUB

ZR = 624
ZR_TAIL = N_NODES - NS * ZR


def _make_segsum():
    mesh = plsc.VectorSubcoreMesh(core_axis_name="c", subcore_axis_name="s")

    @functools.partial(
        pl.kernel,
        mesh=mesh,
        out_type=jax.ShapeDtypeStruct((NC, N_NODES, D), jnp.float32),
        scratch_types=[
            pltpu.VMEM((CH, D), jnp.float32),
            pltpu.VMEM((NSUB, SUB), jnp.int32),
            pltpu.VMEM_SHARED((N_NODES, D), jnp.float32),
            pltpu.SemaphoreType.DMA,
            pltpu.SemaphoreType.DMA,
        ],
    )
    def segsum(edge_hbm, col_hbm, zero_hbm, out_hbm, rows_v, idx_v, acc_s,
               sem_r, sem_i):
        c = lax.axis_index("c")
        s = lax.axis_index("s")
        wid = c * NS + s
        base = wid * EPW

        pltpu.sync_copy(zero_hbm.at[pl.ds(s * ZR, ZR)],
                        acc_s.at[pl.ds(s * ZR, ZR)])

        @pl.when(s == NS - 1)
        def _():
            pltpu.sync_copy(zero_hbm.at[pl.ds(NS * ZR, ZR_TAIL)],
                            acc_s.at[pl.ds(NS * ZR, ZR_TAIL)])

        plsc.subcore_barrier()

        def body(k, carry):
            off = base + k * CH
            cp_r = pltpu.async_copy(edge_hbm.at[pl.ds(off, CH)], rows_v, sem_r)
            cp_i = pltpu.async_copy(col_hbm.at[wid, k], idx_v, sem_i)
            cp_i.wait()
            cp_r.wait()
            for j in range(NSUB):
                pltpu.sync_copy(rows_v.at[pl.ds(j * SUB, SUB)],
                                acc_s.at[idx_v.at[j]],
                                add=True)
            return carry

        lax.fori_loop(0, NCH, body, 0)

        plsc.subcore_barrier()

        pltpu.sync_copy(acc_s.at[pl.ds(s * ZR, ZR)],
                        out_hbm.at[c, pl.ds(s * ZR, ZR)])

        @pl.when(s == NS - 1)
        def _():
            pltpu.sync_copy(acc_s.at[pl.ds(NS * ZR, ZR_TAIL)],
                            out_hbm.at[c, pl.ds(NS * ZR, ZR_TAIL)])

    return segsum


_SEGSUM = _make_segsum()

BLK = 2000


def _mlp_body(x_ref, e_ref, p0_ref, p1_ref, w1_ref, b1_ref, w2_ref, b2_ref,
              o_ref):
    recv = p0_ref[...] + p1_ref[...]
    cat = jnp.concatenate([x_ref[...], e_ref[...], recv], axis=1)
    h = jnp.dot(cat, w1_ref[...], preferred_element_type=jnp.float32)
    h = jnp.maximum(h + b1_ref[...], 0.0)
    h = jnp.dot(h, w2_ref[...], preferred_element_type=jnp.float32)
    o_ref[...] = jnp.maximum(h + b2_ref[...], 0.0)


def _mlp(x, enc, p0, p1, W1, b1, W2, b2):
    rowspec = pl.BlockSpec((BLK, D), lambda i: (i, 0))

    def fullspec(shape):
        return pl.BlockSpec(shape, lambda i: (0,) * len(shape))

    return pl.pallas_call(
        _mlp_body,
        grid=(N_NODES // BLK,),
        in_specs=[rowspec, rowspec, rowspec, rowspec,
                  fullspec((3 * D, D)), fullspec((1, D)),
                  fullspec((D, D)), fullspec((1, D))],
        out_specs=rowspec,
        out_shape=jax.ShapeDtypeStruct((N_NODES, D), jnp.float32),
    )(x, enc, p0, p1, W1, b1.reshape(1, D), W2, b2.reshape(1, D))


def kernel(x, edge_index, edge_attr, encoded_x, batch, W1, b1, W2, b2):
    col = edge_index[1].astype(jnp.int32)
    col4 = col.reshape(NW, NCH, NSUB, SUB)
    zeros = jnp.zeros((N_NODES, D), jnp.float32)
    partial = _SEGSUM(edge_attr, col4, zeros)
    return _mlp(x, encoded_x, partial[0], partial[1], W1, b1, W2, b2)


# speedup vs baseline: 6.5498x; 1.2576x over previous
import functools

import jax
import jax.numpy as jnp
from jax import lax
from jax.experimental import pallas as pl
from jax.experimental.pallas import tpu as pltpu
from jax.experimental.pallas import tpu_sc as plsc

N_NODES = 10000
N_EDGES = 320000
D = 128

NC = 2
NS = 16
NW = NC * NS
EPW = N_EDGES // NW
CHK = 80
NSC = EPW // CHK
NBUF = 3
NLOOP = (NSC - 2) // 3
NREM = NSC - 3 * NLOOP

ZR = 624
ZR_TAIL = N_NODES - NS * ZR


def _make_segsum():
    mesh = plsc.VectorSubcoreMesh(core_axis_name="c", subcore_axis_name="s")

    @functools.partial(
        pl.kernel,
        mesh=mesh,
        out_type=jax.ShapeDtypeStruct((NC, N_NODES, D), jnp.float32),
        scratch_types=[
            pltpu.VMEM((NBUF, CHK, D), jnp.float32),
            pltpu.VMEM((NBUF, CHK), jnp.int32),
            pltpu.VMEM_SHARED((N_NODES, D), jnp.float32),
            pltpu.SemaphoreType.DMA,
            pltpu.SemaphoreType.DMA,
            pltpu.SemaphoreType.DMA,
            pltpu.SemaphoreType.DMA,
            pltpu.SemaphoreType.DMA,
            pltpu.SemaphoreType.DMA,
        ],
    )
    def segsum(edge_hbm, col_hbm, zero_hbm, out_hbm, rows_v, idx_v, acc_s,
               sr0, sr1, sr2, si0, si1, si2):
        sems_r = (sr0, sr1, sr2)
        sems_i = (si0, si1, si2)
        c = lax.axis_index("c")
        s = lax.axis_index("s")
        wid = c * NS + s
        base = wid * EPW

        def issue(k, b):
            off = base + k * CHK
            pltpu.async_copy(edge_hbm.at[pl.ds(off, CHK)], rows_v.at[b],
                             sems_r[b])
            pltpu.async_copy(col_hbm.at[wid, k], idx_v.at[b], sems_i[b])

        def wait_and_scatter(b):
            pltpu.make_async_copy(edge_hbm.at[pl.ds(base, CHK)], rows_v.at[b],
                                  sems_r[b]).wait()
            pltpu.make_async_copy(col_hbm.at[wid, 0], idx_v.at[b],
                                  sems_i[b]).wait()
            pltpu.sync_copy(rows_v.at[b], acc_s.at[idx_v.at[b]], add=True)

        issue(0, 0)
        issue(1, 1)

        pltpu.sync_copy(zero_hbm.at[pl.ds(s * ZR, ZR)],
                        acc_s.at[pl.ds(s * ZR, ZR)])

        @pl.when(s == NS - 1)
        def _():
            pltpu.sync_copy(zero_hbm.at[pl.ds(NS * ZR, ZR_TAIL)],
                            acc_s.at[pl.ds(NS * ZR, ZR_TAIL)])

        plsc.subcore_barrier()

        def body(i, carry):
            k = 3 * i
            for t in range(NBUF):
                wait_and_scatter(t)
                issue(k + t + 2, (t + 2) % NBUF)
            return carry

        lax.fori_loop(0, NLOOP, body, 0)

        for t in range(NREM):
            k = 3 * NLOOP + t
            wait_and_scatter(k % NBUF)
            if k + 2 < NSC:
                issue(k + 2, (k + 2) % NBUF)

        plsc.subcore_barrier()

        pltpu.sync_copy(acc_s.at[pl.ds(s * ZR, ZR)],
                        out_hbm.at[c, pl.ds(s * ZR, ZR)])

        @pl.when(s == NS - 1)
        def _():
            pltpu.sync_copy(acc_s.at[pl.ds(NS * ZR, ZR_TAIL)],
                            out_hbm.at[c, pl.ds(NS * ZR, ZR_TAIL)])

    return segsum


_SEGSUM = _make_segsum()

BLK = 2000


def _mlp_body(x_ref, e_ref, p0_ref, p1_ref, w1_ref, b1_ref, w2_ref, b2_ref,
              o_ref):
    recv = p0_ref[...] + p1_ref[...]
    cat = jnp.concatenate([x_ref[...], e_ref[...], recv], axis=1)
    h = jnp.dot(cat, w1_ref[...], preferred_element_type=jnp.float32)
    h = jnp.maximum(h + b1_ref[...], 0.0)
    h = jnp.dot(h, w2_ref[...], preferred_element_type=jnp.float32)
    o_ref[...] = jnp.maximum(h + b2_ref[...], 0.0)


def _mlp(x, enc, p0, p1, W1, b1, W2, b2):
    rowspec = pl.BlockSpec((BLK, D), lambda i: (i, 0))

    def fullspec(shape):
        return pl.BlockSpec(shape, lambda i: (0,) * len(shape))

    return pl.pallas_call(
        _mlp_body,
        grid=(N_NODES // BLK,),
        in_specs=[rowspec, rowspec, rowspec, rowspec,
                  fullspec((3 * D, D)), fullspec((1, D)),
                  fullspec((D, D)), fullspec((1, D))],
        out_specs=rowspec,
        out_shape=jax.ShapeDtypeStruct((N_NODES, D), jnp.float32),
    )(x, enc, p0, p1, W1, b1.reshape(1, D), W2, b2.reshape(1, D))


def kernel(x, edge_index, edge_attr, encoded_x, batch, W1, b1, W2, b2):
    col = edge_index[1].astype(jnp.int32)
    col4 = col.reshape(NW, NSC, CHK)
    zeros = jnp.zeros((N_NODES, D), jnp.float32)
    partial = _SEGSUM(edge_attr, col4, zeros)
    return _mlp(x, encoded_x, partial[0], partial[1], W1, b1, W2, b2)


# speedup vs baseline: 7.3924x; 1.1286x over previous
import functools

import jax
import jax.numpy as jnp
from jax import lax
from jax.experimental import pallas as pl
from jax.experimental.pallas import tpu as pltpu
from jax.experimental.pallas import tpu_sc as plsc

N_NODES = 10000
N_EDGES = 320000
D = 128

NC = 2
NS = 16
NW = NC * NS
EPW = N_EDGES // NW
CHK = 128
NSC = -(-EPW // CHK)
NTAIL = EPW - (NSC - 1) * CHK
N_PAD = N_NODES + 8
NBUF = 3
NLOOP = (NSC - 2) // 3
NREM = NSC - 3 * NLOOP

ZR = 624
ZR_TAIL = N_NODES - NS * ZR


def _make_segsum():
    mesh = plsc.VectorSubcoreMesh(core_axis_name="c", subcore_axis_name="s")

    @functools.partial(
        pl.kernel,
        mesh=mesh,
        out_type=jax.ShapeDtypeStruct((NC, N_NODES, D), jnp.float32),
        scratch_types=[
            pltpu.VMEM((NBUF, CHK, D), jnp.float32),
            pltpu.VMEM((NBUF, CHK), jnp.int32),
            pltpu.VMEM_SHARED((N_PAD, D), jnp.float32),
            pltpu.SemaphoreType.DMA,
            pltpu.SemaphoreType.DMA,
            pltpu.SemaphoreType.DMA,
            pltpu.SemaphoreType.DMA,
            pltpu.SemaphoreType.DMA,
            pltpu.SemaphoreType.DMA,
        ],
    )
    def segsum(edge_hbm, col_hbm, zero_hbm, out_hbm, rows_v, idx_v, acc_s,
               sr0, sr1, sr2, si0, si1, si2):
        sems_r = (sr0, sr1, sr2)
        sems_i = (si0, si1, si2)
        c = lax.axis_index("c")
        s = lax.axis_index("s")
        wid = c * NS + s
        base = wid * EPW

        cbase = wid * (NSC * CHK)

        def issue(k, b, last=False):
            off = base + (EPW - CHK if last else k * CHK)
            pltpu.async_copy(edge_hbm.at[pl.ds(off, CHK)], rows_v.at[b],
                             sems_r[b])
            pltpu.async_copy(col_hbm.at[pl.ds(cbase + k * CHK, CHK)],
                             idx_v.at[b], sems_i[b])

        def wait_and_scatter(b):
            pltpu.make_async_copy(edge_hbm.at[pl.ds(base, CHK)], rows_v.at[b],
                                  sems_r[b]).wait()
            pltpu.make_async_copy(col_hbm.at[pl.ds(cbase, CHK)], idx_v.at[b],
                                  sems_i[b]).wait()
            pltpu.sync_copy(rows_v.at[b], acc_s.at[idx_v.at[b]], add=True)

        issue(0, 0)
        issue(1, 1)

        pltpu.sync_copy(zero_hbm.at[pl.ds(0, ZR)], acc_s.at[pl.ds(s * ZR, ZR)])

        @pl.when(s == NS - 1)
        def _():
            pltpu.sync_copy(zero_hbm.at[pl.ds(0, ZR_TAIL)],
                            acc_s.at[pl.ds(NS * ZR, ZR_TAIL)])

        plsc.subcore_barrier()

        def body(i, carry):
            k = 3 * i
            for t in range(NBUF):
                wait_and_scatter(t)
                issue(k + t + 2, (t + 2) % NBUF)
            return carry

        lax.fori_loop(0, NLOOP, body, 0)

        for t in range(NREM):
            k = 3 * NLOOP + t
            wait_and_scatter(k % NBUF)
            nxt = k + 2
            if nxt < NSC:
                issue(nxt, nxt % NBUF, last=(nxt == NSC - 1))

        plsc.subcore_barrier()

        pltpu.sync_copy(acc_s.at[pl.ds(s * ZR, ZR)],
                        out_hbm.at[c, pl.ds(s * ZR, ZR)])

        @pl.when(s == NS - 1)
        def _():
            pltpu.sync_copy(acc_s.at[pl.ds(NS * ZR, ZR_TAIL)],
                            out_hbm.at[c, pl.ds(NS * ZR, ZR_TAIL)])

    return segsum


_SEGSUM = _make_segsum()

BLK = 2000


def _mlp_body(x_ref, e_ref, p0_ref, p1_ref, w1_ref, b1_ref, w2_ref, b2_ref,
              o_ref):
    recv = p0_ref[...] + p1_ref[...]
    cat = jnp.concatenate([x_ref[...], e_ref[...], recv], axis=1)
    h = jnp.dot(cat, w1_ref[...], preferred_element_type=jnp.float32)
    h = jnp.maximum(h + b1_ref[...], 0.0)
    h = jnp.dot(h, w2_ref[...], preferred_element_type=jnp.float32)
    o_ref[...] = jnp.maximum(h + b2_ref[...], 0.0)


def _mlp(x, enc, partial, W1, b1, W2, b2):
    rowspec = pl.BlockSpec((BLK, D), lambda i: (i, 0))

    def fullspec(shape):
        return pl.BlockSpec(shape, lambda i: (0,) * len(shape))

    pflat = partial.reshape(NC * N_NODES, D)
    p1spec = pl.BlockSpec((BLK, D), lambda i: (i + N_NODES // BLK, 0))

    return pl.pallas_call(
        _mlp_body,
        grid=(N_NODES // BLK,),
        in_specs=[rowspec, rowspec, rowspec, p1spec,
                  fullspec((3 * D, D)), fullspec((1, D)),
                  fullspec((D, D)), fullspec((1, D))],
        out_specs=rowspec,
        out_shape=jax.ShapeDtypeStruct((N_NODES, D), jnp.float32),
    )(x, enc, pflat, pflat, W1, b1.reshape(1, D), W2, b2.reshape(1, D))


def kernel(x, edge_index, edge_attr, encoded_x, batch, W1, b1, W2, b2):
    col = edge_index[1].astype(jnp.int32).reshape(NW, EPW)
    full = col[:, :(NSC - 1) * CHK].reshape(NW, NSC - 1, CHK)
    tail = jnp.concatenate(
        [jnp.full((NW, CHK - NTAIL), N_NODES, jnp.int32),
         col[:, EPW - NTAIL:]], axis=1)
    col3 = jnp.concatenate([full, tail[:, None, :]], axis=1).reshape(-1)
    zeros = jnp.zeros((ZR, D), jnp.float32)
    partial = _SEGSUM(edge_attr, col3, zeros)
    return _mlp(x, encoded_x, partial, W1, b1, W2, b2)
